# baseline (speedup 1.0000x reference)
import functools

import jax
import jax.numpy as jnp
from jax import lax
from jax.experimental import pallas as pl
from jax.experimental.pallas import tpu as pltpu
from jax.experimental.pallas import tpu_sc as plsc

_VOCAB = 1000000
_DIM = 64
_B = 16384
_K = 20

_NW = 32
_NB = _B // _NW
_BLK = 16
_NBLK = _NB // _BLK
_NEG_BLK = _BLK * _K
_ICH = 64
_NCH = _NEG_BLK // _ICH

_VR = 500000
_FULL_BLOCKS = _VOCAB // 128
_TAIL = _VOCAB - _FULL_BLOCKS * 128
_EXTRA = _FULL_BLOCKS - (_FULL_BLOCKS // _NW) * _NW

_SC_PARAMS = pltpu.CompilerParams(
    needs_layout_passes=False, use_tc_tiling_on_sc=True)
_MESH = plsc.VectorSubcoreMesh(core_axis_name="c", subcore_axis_name="s")


def _format_body(twt, cwt, tw2, cw2, buf, ostg, buf64, ostg64):
    wid = lax.axis_index("s") * 2 + lax.axis_index("c")
    iotas = [lax.iota(jnp.int32, 16) + q * 16 for q in range(4)]

    def do_table(src, dst):
        nb = jnp.where(wid < _EXTRA, _FULL_BLOCKS // _NW + 1,
                       _FULL_BLOCKS // _NW)

        def blk(it, c):
            b = wid + it * _NW
            pltpu.sync_copy(src.at[:, pl.ds(pl.multiple_of(b * 128, 128), 128)], buf)

            def row(i2, c2):
                for q in range(8):
                    jv = jnp.full((16,), 2 * i2 + q // 4, jnp.int32)
                    v = plsc.load_gather(buf, [iotas[q % 4], jv])
                    ostg[i2, pl.ds(q * 16, 16)] = v
                return c2

            lax.fori_loop(0, 64, row, 0)
            pltpu.sync_copy(ostg, dst.at[pl.ds(pl.multiple_of(b * 64, 64), 64), :])
            return c

        lax.fori_loop(0, nb, blk, 0)

    do_table(twt, tw2)
    do_table(cwt, cw2)

    @pl.when(wid == _EXTRA)
    def _():
        for src, dst in ((twt, tw2), (cwt, cw2)):
            pltpu.sync_copy(src.at[:, pl.ds(_FULL_BLOCKS * 128, _TAIL)], buf64)

            def row(i2, c2):
                for q in range(8):
                    jv = jnp.full((16,), 2 * i2 + q // 4, jnp.int32)
                    v = plsc.load_gather(buf64, [iotas[q % 4], jv])
                    ostg64[i2, pl.ds(q * 16, 16)] = v
                return c2

            lax.fori_loop(0, _TAIL // 2, row, 0)
            pltpu.sync_copy(
                ostg64, dst.at[pl.ds(_FULL_BLOCKS * 64, _TAIL // 2), :])


_sc_format = functools.partial(
    pl.kernel,
    out_type=(
        jax.ShapeDtypeStruct((_VR, 128), jnp.float32),
        jax.ShapeDtypeStruct((_VR, 128), jnp.float32),
    ),
    mesh=_MESH,
    scratch_types=[
        pltpu.VMEM((_DIM, 128), jnp.float32),
        pltpu.VMEM((64, 128), jnp.float32),
        pltpu.VMEM((_DIM, _TAIL), jnp.float32),
        pltpu.VMEM((_TAIL // 2, 128), jnp.float32),
    ],
    compiler_params=_SC_PARAMS,
)(_format_body)


def _scores_body(th, ph, nh, traw, praw, nraw, tw2, cw2,
                 pos_out, neg_out,
                 tih, pih, nih, ts_s, ps_s, ns_s, t_v, p_v, n_v,
                 psc, nsc, sem):
    wid = lax.axis_index("s") * 2 + lax.axis_index("c")
    iota16 = lax.iota(jnp.int32, 16)
    zeros16 = jnp.zeros((16,), jnp.float32)

    pltpu.sync_copy(th.at[wid], tih)
    pltpu.sync_copy(ph.at[wid], pih)
    pltpu.sync_copy(nh.at[wid], nih)
    pltpu.sync_copy(traw.at[wid], ts_s)
    pltpu.sync_copy(praw.at[wid], ps_s)
    pltpu.sync_copy(nraw.at[wid], ns_s)

    def blk_body(blk, carry):
        cps = [
            pltpu.async_copy(tw2.at[tih.at[blk]], t_v, sem),
            pltpu.async_copy(cw2.at[pih.at[blk]], p_v, sem),
        ]
        for j in range(_NCH):
            cps.append(pltpu.async_copy(
                cw2.at[nih.at[blk * _NCH + j]],
                n_v.at[pl.ds(j * _ICH, _ICH)], sem))
        for cp in cps:
            cp.wait()

        tpar = (ts_s[blk, :] & 1) * 64
        ppar = (ps_s[blk, :] & 1) * 64
        accp = zeros16
        for u in range(_BLK // 4):
            teffs = []
            for uj in range(4):
                i = u * 4 + uj
                toff = tpar[i]
                te = [t_v[i, pl.ds(toff + 16 * cc, 16)] for cc in range(4)]
                teffs.append(te)
                poff = ppar[i]
                pe = [p_v[i, pl.ds(poff + 16 * cc, 16)] for cc in range(4)]
                s = jnp.sum(te[0] * pe[0] + te[1] * pe[1]
                            + te[2] * pe[2] + te[3] * pe[3])
                accp = jnp.where(iota16 == i, s, accp)
            out_base = blk * _NEG_BLK + u * 80
            for v5 in range(5):
                npar = (ns_s[blk, pl.ds(u * 80 + v5 * 16, 16)] & 1) * 64
                accn = zeros16
                for l in range(16):
                    d = v5 * 16 + l
                    te = teffs[d // _K]
                    f = u * 80 + d
                    noff = npar[l]
                    nr = [n_v[f, pl.ds(noff + 16 * cc, 16)] for cc in range(4)]
                    s = jnp.sum(nr[0] * te[0] + nr[1] * te[1]
                                + nr[2] * te[2] + nr[3] * te[3])
                    accn = jnp.where(iota16 == l, s, accn)
                nsc[pl.ds(out_base + v5 * 16, 16)] = accn
        psc[pl.ds(blk * _BLK, 16)] = accp
        return carry

    lax.fori_loop(0, _NBLK, blk_body, 0)

    pltpu.sync_copy(psc, pos_out.at[wid])
    pltpu.sync_copy(nsc, neg_out.at[wid])


_sc_scores = functools.partial(
    pl.kernel,
    out_type=(
        jax.ShapeDtypeStruct((_NW, _NB), jnp.float32),
        jax.ShapeDtypeStruct((_NW, _NB * _K), jnp.float32),
    ),
    mesh=_MESH,
    scratch_types=[
        pltpu.VMEM((_NBLK, _BLK), jnp.int32),
        pltpu.VMEM((_NBLK, _BLK), jnp.int32),
        pltpu.VMEM((_NBLK * _NCH, _ICH), jnp.int32),
        pltpu.VMEM((_NBLK, _BLK), jnp.int32),
        pltpu.VMEM((_NBLK, _BLK), jnp.int32),
        pltpu.VMEM((_NBLK, _NEG_BLK), jnp.int32),
        pltpu.VMEM((_BLK, 128), jnp.float32),
        pltpu.VMEM((_BLK, 128), jnp.float32),
        pltpu.VMEM((_NEG_BLK, 128), jnp.float32),
        pltpu.VMEM((_NB,), jnp.float32),
        pltpu.VMEM((_NB * _K,), jnp.float32),
        pltpu.SemaphoreType.DMA,
    ],
    compiler_params=_SC_PARAMS,
)(_scores_body)


def _tc_loss_body(ps_ref, ns_ref, o_ref):
    def logsig(x):
        return jnp.minimum(x, 0.0) - jnp.log1p(jnp.exp(-jnp.abs(x)))

    ps = ps_ref[...]
    ns = ns_ref[...]
    pos_loss = -jnp.sum(jnp.sum(logsig(ps), axis=0)) / _B
    neg_loss = -jnp.sum(jnp.sum(logsig(-ns), axis=0)) / (_B * _K)
    o_ref[0, 0] = pos_loss + neg_loss


_tc_loss = pl.pallas_call(
    _tc_loss_body,
    out_shape=jax.ShapeDtypeStruct((1, 1), jnp.float32),
    out_specs=pl.BlockSpec(memory_space=pltpu.SMEM),
)


def kernel(target_ids, pos_ids, neg_ids, target_w, context_w):
    tw2, cw2 = _sc_format(target_w.T, context_w.T)
    traw = target_ids.astype(jnp.int32).reshape(_NW, _NBLK, _BLK)
    praw = pos_ids.astype(jnp.int32).reshape(_NW, _NBLK, _BLK)
    nraw = neg_ids.astype(jnp.int32).reshape(_NW, _NBLK, _NEG_BLK)
    th = traw >> 1
    ph = praw >> 1
    nh = (neg_ids.astype(jnp.int32) >> 1).reshape(_NW, _NBLK * _NCH, _ICH)
    ps, ns = _sc_scores(th, ph, nh, traw, praw, nraw, tw2, cw2)
    loss = _tc_loss(ps.reshape(128, 128), ns.reshape(2560, 128))
    return loss[0, 0]


# speedup vs baseline: 1.2182x; 1.2182x over previous
import functools

import jax
import jax.numpy as jnp
from jax import lax
from jax.experimental import pallas as pl
from jax.experimental.pallas import tpu as pltpu
from jax.experimental.pallas import tpu_sc as plsc

_VOCAB = 1000000
_DIM = 64
_B = 16384
_K = 20

_NW = 32
_NB = _B // _NW
_BLK = 16
_NBLK = _NB // _BLK
_NEG_BLK = _BLK * _K
_ICH = 64
_NCH = _NEG_BLK // _ICH

_VR = 500000
_SB = 384
_FULL_BLOCKS = _VOCAB // _SB
_TAIL = _VOCAB - _FULL_BLOCKS * _SB
_EXTRA = _FULL_BLOCKS - (_FULL_BLOCKS // _NW) * _NW

_SC_PARAMS = pltpu.CompilerParams(
    needs_layout_passes=False, use_tc_tiling_on_sc=True)
_MESH = plsc.VectorSubcoreMesh(core_axis_name="c", subcore_axis_name="s")


def _format_body(twt, cwt, tw2, cw2, buf, ostg, buf64, ostg64, sem_i, sem_o):
    wid = lax.axis_index("s") * 2 + lax.axis_index("c")
    iotas = [lax.iota(jnp.int32, 16) + q * 16 for q in range(4)]

    def do_table(src, dst):
        nb = jnp.where(wid < _EXTRA, _FULL_BLOCKS // _NW + 1,
                       _FULL_BLOCKS // _NW)

        def in_slice(it):
            b = wid + it * _NW
            return src.at[:, pl.ds(pl.multiple_of(b * _SB, 128), _SB)]

        def out_slice(it):
            b = wid + it * _NW
            return dst.at[pl.ds(pl.multiple_of(b * (_SB // 2), 64), _SB // 2), :]

        pltpu.async_copy(in_slice(0), buf.at[0], sem_i)

        def blk(it, c):
            cur = it & 1
            pltpu.make_async_copy(in_slice(it), buf.at[cur], sem_i).wait()

            @pl.when(it + 1 < nb)
            def _():
                pltpu.async_copy(in_slice(it + 1), buf.at[1 - cur], sem_i)

            @pl.when(it >= 2)
            def _():
                pltpu.make_async_copy(
                    ostg.at[cur], out_slice(it - 2), sem_o).wait()

            def row(r4, c2):
                for rr in range(4):
                    i2 = r4 * 4 + rr
                    for q in range(8):
                        jv = jnp.full((16,), 2 * i2 + q // 4, jnp.int32)
                        v = plsc.load_gather(buf.at[cur], [iotas[q % 4], jv])
                        ostg[cur, i2, pl.ds(q * 16, 16)] = v
                return c2

            lax.fori_loop(0, _SB // 8, row, 0)
            pltpu.async_copy(ostg.at[cur], out_slice(it), sem_o)
            return c

        lax.fori_loop(0, nb, blk, 0)

        @pl.when(nb >= 2)
        def _():
            pltpu.make_async_copy(
                ostg.at[nb & 1], out_slice(nb - 2), sem_o).wait()
        pltpu.make_async_copy(
            ostg.at[(nb - 1) & 1], out_slice(nb - 1), sem_o).wait()

    do_table(twt, tw2)
    do_table(cwt, cw2)

    @pl.when(wid == _EXTRA)
    def _():
        for src, dst in ((twt, tw2), (cwt, cw2)):
            pltpu.sync_copy(src.at[:, pl.ds(_FULL_BLOCKS * _SB, _TAIL)], buf64)

            def row(i2, c2):
                for q in range(8):
                    jv = jnp.full((16,), 2 * i2 + q // 4, jnp.int32)
                    v = plsc.load_gather(buf64, [iotas[q % 4], jv])
                    ostg64[i2, pl.ds(q * 16, 16)] = v
                return c2

            lax.fori_loop(0, _TAIL // 2, row, 0)
            pltpu.sync_copy(
                ostg64, dst.at[pl.ds(_VOCAB // 2 - _TAIL // 2, _TAIL // 2), :])


_sc_format = functools.partial(
    pl.kernel,
    out_type=(
        jax.ShapeDtypeStruct((_VR, 128), jnp.float32),
        jax.ShapeDtypeStruct((_VR, 128), jnp.float32),
    ),
    mesh=_MESH,
    scratch_types=[
        pltpu.VMEM((2, _DIM, _SB), jnp.float32),
        pltpu.VMEM((2, _SB // 2, 128), jnp.float32),
        pltpu.VMEM((_DIM, _TAIL), jnp.float32),
        pltpu.VMEM((_TAIL // 2, 128), jnp.float32),
        pltpu.SemaphoreType.DMA,
        pltpu.SemaphoreType.DMA,
    ],
    compiler_params=_SC_PARAMS,
)(_format_body)


def _scores_body(th, ph, nh, traw, praw, nraw, tw2, cw2,
                 pos_out, neg_out,
                 tih, pih, nih, ts_s, ps_s, ns_s, t_v, p_v, n_v,
                 psc, nsc, sem):
    wid = lax.axis_index("s") * 2 + lax.axis_index("c")
    iota16 = lax.iota(jnp.int32, 16)
    zeros16 = jnp.zeros((16,), jnp.float32)

    pltpu.sync_copy(th.at[wid], tih)
    pltpu.sync_copy(ph.at[wid], pih)
    pltpu.sync_copy(nh.at[wid], nih)
    pltpu.sync_copy(traw.at[wid], ts_s)
    pltpu.sync_copy(praw.at[wid], ps_s)
    pltpu.sync_copy(nraw.at[wid], ns_s)

    def blk_body(blk, carry):
        cps = [
            pltpu.async_copy(tw2.at[tih.at[blk]], t_v, sem),
            pltpu.async_copy(cw2.at[pih.at[blk]], p_v, sem),
        ]
        for j in range(_NCH):
            cps.append(pltpu.async_copy(
                cw2.at[nih.at[blk * _NCH + j]],
                n_v.at[pl.ds(j * _ICH, _ICH)], sem))
        for cp in cps:
            cp.wait()

        tpar = (ts_s[blk, :] & 1) * 64
        ppar = (ps_s[blk, :] & 1) * 64
        accp = zeros16
        for u in range(_BLK // 4):
            teffs = []
            for uj in range(4):
                i = u * 4 + uj
                toff = tpar[i]
                te = [t_v[i, pl.ds(toff + 16 * cc, 16)] for cc in range(4)]
                teffs.append(te)
                poff = ppar[i]
                pe = [p_v[i, pl.ds(poff + 16 * cc, 16)] for cc in range(4)]
                s = jnp.sum(te[0] * pe[0] + te[1] * pe[1]
                            + te[2] * pe[2] + te[3] * pe[3])
                accp = jnp.where(iota16 == i, s, accp)
            out_base = blk * _NEG_BLK + u * 80
            for v5 in range(5):
                npar = (ns_s[blk, pl.ds(u * 80 + v5 * 16, 16)] & 1) * 64
                accn = zeros16
                for l in range(16):
                    d = v5 * 16 + l
                    te = teffs[d // _K]
                    f = u * 80 + d
                    noff = npar[l]
                    nr = [n_v[f, pl.ds(noff + 16 * cc, 16)] for cc in range(4)]
                    s = jnp.sum(nr[0] * te[0] + nr[1] * te[1]
                                + nr[2] * te[2] + nr[3] * te[3])
                    accn = jnp.where(iota16 == l, s, accn)
                nsc[pl.ds(out_base + v5 * 16, 16)] = accn
        psc[pl.ds(blk * _BLK, 16)] = accp
        return carry

    lax.fori_loop(0, _NBLK, blk_body, 0)

    pltpu.sync_copy(psc, pos_out.at[wid])
    pltpu.sync_copy(nsc, neg_out.at[wid])


_sc_scores = functools.partial(
    pl.kernel,
    out_type=(
        jax.ShapeDtypeStruct((_NW, _NB), jnp.float32),
        jax.ShapeDtypeStruct((_NW, _NB * _K), jnp.float32),
    ),
    mesh=_MESH,
    scratch_types=[
        pltpu.VMEM((_NBLK, _BLK), jnp.int32),
        pltpu.VMEM((_NBLK, _BLK), jnp.int32),
        pltpu.VMEM((_NBLK * _NCH, _ICH), jnp.int32),
        pltpu.VMEM((_NBLK, _BLK), jnp.int32),
        pltpu.VMEM((_NBLK, _BLK), jnp.int32),
        pltpu.VMEM((_NBLK, _NEG_BLK), jnp.int32),
        pltpu.VMEM((_BLK, 128), jnp.float32),
        pltpu.VMEM((_BLK, 128), jnp.float32),
        pltpu.VMEM((_NEG_BLK, 128), jnp.float32),
        pltpu.VMEM((_NB,), jnp.float32),
        pltpu.VMEM((_NB * _K,), jnp.float32),
        pltpu.SemaphoreType.DMA,
    ],
    compiler_params=_SC_PARAMS,
)(_scores_body)


def _tc_loss_body(ps_ref, ns_ref, o_ref):
    def logsig(x):
        return jnp.minimum(x, 0.0) - jnp.log1p(jnp.exp(-jnp.abs(x)))

    ps = ps_ref[...]
    ns = ns_ref[...]
    pos_loss = -jnp.sum(jnp.sum(logsig(ps), axis=0)) / _B
    neg_loss = -jnp.sum(jnp.sum(logsig(-ns), axis=0)) / (_B * _K)
    o_ref[0, 0] = pos_loss + neg_loss


_tc_loss = pl.pallas_call(
    _tc_loss_body,
    out_shape=jax.ShapeDtypeStruct((1, 1), jnp.float32),
    out_specs=pl.BlockSpec(memory_space=pltpu.SMEM),
)


def kernel(target_ids, pos_ids, neg_ids, target_w, context_w):
    tw2, cw2 = _sc_format(target_w.T, context_w.T)
    traw = target_ids.astype(jnp.int32).reshape(_NW, _NBLK, _BLK)
    praw = pos_ids.astype(jnp.int32).reshape(_NW, _NBLK, _BLK)
    nraw = neg_ids.astype(jnp.int32).reshape(_NW, _NBLK, _NEG_BLK)
    th = traw >> 1
    ph = praw >> 1
    nh = (neg_ids.astype(jnp.int32) >> 1).reshape(_NW, _NBLK * _NCH, _ICH)
    ps, ns = _sc_scores(th, ph, nh, traw, praw, nraw, tw2, cw2)
    loss = _tc_loss(ps.reshape(128, 128), ns.reshape(2560, 128))
    return loss[0, 0]
